# baseline (speedup 1.0000x reference)
import dataclasses
import functools

import jax
import jax.numpy as jnp
from jax import lax
from jax.experimental import pallas as pl
from jax.experimental.pallas import tpu as pltpu
from jax.experimental.pallas import tpu_sc as plsc

_NC = 2
_NS = 16
_NW = _NC * _NS
_L = 16
_G = 128
_GPC = 3
_W = _G * _GPC


def _take16(c, idx):
    dnums = lax.GatherDimensionNumbers(
        offset_dims=(), collapsed_slice_dims=(0,), start_index_map=(0,))
    return lax.gather(c, idx[:, None], dnums, slice_sizes=(1,),
                      mode=lax.GatherScatterMode.PROMISE_IN_BOUNDS)


def _sc_kernel_body(node_hbm, row_hbm, col_hbm, out_hbm,
                    ntab, idx_a0, idx_a1, idx_b0, idx_b1,
                    arows0, arows1, brows0, brows1, out_v0, out_v1,
                    sem_i0, sem_i1, sem_g0, sem_g1, sem_h0, sem_h1,
                    sem_o0, sem_o1):
    n_chunks = row_hbm.shape[0] // (_NW * _W)
    sid = lax.axis_index("s")
    wid = sid * _NC + lax.axis_index("c")
    base_e = wid * n_chunks * _W

    n_nodes = node_hbm.shape[0]
    rows_per_tile = n_nodes // _NS
    pltpu.sync_copy(node_hbm.at[pl.ds(sid * rows_per_tile, rows_per_tile)],
                    ntab.at[pl.ds(sid * rows_per_tile, rows_per_tile)])
    plsc.subcore_barrier()

    idx_a = (idx_a0, idx_a1)
    idx_b = (idx_b0, idx_b1)
    arows = (arows0, arows1)
    brows = (brows0, brows1)
    out_v = (out_v0, out_v1)
    sem_i = (sem_i0, sem_i1)
    sem_g = (sem_g0, sem_g1)
    sem_h = (sem_h0, sem_h1)
    sem_o = (sem_o0, sem_o1)

    lane = lax.iota(jnp.int32, _L)
    sstart = ((lane >= 1).astype(jnp.int32) + 3 * (lane >= 4).astype(jnp.int32)
              + 5 * (lane >= 9).astype(jnp.int32))
    i1 = jnp.maximum(lane - 1, 0)
    i2 = jnp.maximum(lane - 2, 0)
    i4 = jnp.maximum(lane - 4, 0)
    m1 = (lane - 1) >= sstart
    m2 = (lane - 2) >= sstart
    m4 = (lane - 4) >= sstart
    q = lane & 3
    fin = q * q + 2 * q
    grp = lane >> 2
    fzero = jnp.float32(0.0)

    def start_idx(k, b):
        e0 = base_e + k * _W
        pltpu.async_copy(row_hbm.at[pl.ds(e0, _W)], idx_a[b], sem_i[b])
        pltpu.async_copy(col_hbm.at[pl.ds(e0, _W)], idx_b[b], sem_i[b])

    def drain_idx(b):
        pltpu.make_async_copy(row_hbm.at[pl.ds(0, _W)], idx_a[b],
                              sem_i[b]).wait()
        pltpu.make_async_copy(col_hbm.at[pl.ds(0, _W)], idx_b[b],
                              sem_i[b]).wait()

    def start_gathers(b):
        for j in range(_GPC):
            pltpu.async_copy(
                ntab.at[idx_a[b].at[pl.ds(j * _G, _G)]],
                arows[b].at[pl.ds(j * _G, _G)], sem_g[b])
            pltpu.async_copy(
                node_hbm.at[idx_b[b].at[pl.ds(j * _G, _G)]],
                brows[b].at[pl.ds(j * _G, _G)], sem_h[b])

    def drain_gathers(b):
        pltpu.make_async_copy(ntab.at[pl.ds(0, _W)], arows[b],
                              sem_g[b]).wait()
        pltpu.make_async_copy(node_hbm.at[pl.ds(0, _W)], brows[b],
                              sem_h[b]).wait()

    def start_out(k, b):
        e0 = base_e + k * _W
        pltpu.async_copy(out_v[b], out_hbm.at[pl.ds(e0 * 4, _W * 4)],
                         sem_o[b])

    def drain_out(b):
        pltpu.make_async_copy(out_v[b], out_hbm.at[pl.ds(0, _W * 4)],
                              sem_o[b]).wait()

    def seg_sums(t):
        x = t + jnp.where(m1, _take16(t, i1), fzero)
        x = x + jnp.where(m2, _take16(x, i2), fzero)
        x = x + jnp.where(m4, _take16(x, i4), fzero)
        return _take16(x, fin)

    def compute(b):
        @plsc.parallel_loop(0, _W, step=4, unroll=8)
        def _edge(e):
            g = [seg_sums(arows[b][e + u, :] * brows[b][e + u, :])
                 for u in range(4)]
            acc = jnp.where(grp == 1, g[1], g[0])
            acc = jnp.where(grp == 2, g[2], acc)
            acc = jnp.where(grp == 3, g[3], acc)
            out_v[b][pl.ds(e * 4, _L)] = acc

    start_idx(0, 0)
    start_idx(1, 1)
    drain_idx(0)
    start_gathers(0)

    @pl.loop(0, n_chunks, step=2)
    def _pair(kk):
        for b in (0, 1):
            k = kk + b
            ob = 1 - b
            drain_gathers(b)

            @pl.when(k + 2 < n_chunks)
            def _():
                start_idx(k + 2, b)

            @pl.when(k + 1 < n_chunks)
            def _():
                drain_idx(ob)
                start_gathers(ob)

            @pl.when(k >= 2)
            def _():
                drain_out(b)

            compute(b)
            start_out(k, b)

    drain_out(0)
    drain_out(1)


@functools.partial(jax.jit, static_argnames=("e_pad",))
def _sc_call(node_sh, row2, col2, e_pad):
    mesh = plsc.VectorSubcoreMesh(core_axis_name="c", subcore_axis_name="s")
    cp = pltpu.CompilerParams()
    if "needs_layout_passes" in pltpu.CompilerParams.__dataclass_fields__:
        cp = dataclasses.replace(cp, needs_layout_passes=False)
    if "use_tc_tiling_on_sc" in pltpu.CompilerParams.__dataclass_fields__:
        cp = dataclasses.replace(cp, use_tc_tiling_on_sc=False)
    kfn = pl.kernel(
        _sc_kernel_body,
        out_type=jax.ShapeDtypeStruct((e_pad * 4,), jnp.float32),
        mesh=mesh,
        scratch_types=(
            [pltpu.VMEM_SHARED(node_sh.shape, jnp.float32)]
            + [pltpu.VMEM((_W,), jnp.int32)] * 4
            + [pltpu.VMEM((_W, _L), jnp.float32)] * 4
            + [pltpu.VMEM((_W * 4,), jnp.float32)] * 2
            + [pltpu.SemaphoreType.DMA] * 8
        ),
        compiler_params=cp,
    )
    return kfn(node_sh, row2, col2)


def kernel(edge_index, node_sh):
    e = edge_index.shape[1]
    per_tile = -(-e // (_NW * _W * 2)) * _W * 2
    e_pad = per_tile * _NW
    ei = edge_index.astype(jnp.int32)
    row = jnp.pad(ei[0], (0, e_pad - e))
    col = jnp.pad(ei[1], (0, e_pad - e))
    out = _sc_call(node_sh, row, col, e_pad)
    return out[:e * 4].reshape(e, 4)


# speedup vs baseline: 1.0028x; 1.0028x over previous
import dataclasses
import functools

import jax
import jax.numpy as jnp
from jax import lax
from jax.experimental import pallas as pl
from jax.experimental.pallas import tpu as pltpu
from jax.experimental.pallas import tpu_sc as plsc

_NC = 2
_NS = 16
_NW = _NC * _NS
_L = 16
_G = 128
_GPC = 3
_W = _G * _GPC


def _take16(c, idx):
    dnums = lax.GatherDimensionNumbers(
        offset_dims=(), collapsed_slice_dims=(0,), start_index_map=(0,))
    return lax.gather(c, idx[:, None], dnums, slice_sizes=(1,),
                      mode=lax.GatherScatterMode.PROMISE_IN_BOUNDS)


def _sc_kernel_body(node_hbm, row_hbm, col_hbm, out_hbm,
                    ntab, idx_a0, idx_a1, idx_b0, idx_b1,
                    arows0, arows1, brows0, brows1, out_v0, out_v1,
                    sem_i0, sem_i1, sem_g0, sem_g1, sem_h0, sem_h1,
                    sem_o0, sem_o1):
    n_chunks = row_hbm.shape[0] // (_NW * _W)
    sid = lax.axis_index("s")
    wid = sid * _NC + lax.axis_index("c")
    base_e = wid * n_chunks * _W

    n_nodes = node_hbm.shape[0]
    rows_per_tile = n_nodes // _NS
    pltpu.sync_copy(node_hbm.at[pl.ds(sid * rows_per_tile, rows_per_tile)],
                    ntab.at[pl.ds(sid * rows_per_tile, rows_per_tile)])
    plsc.subcore_barrier()

    idx_a = (idx_a0, idx_a1)
    idx_b = (idx_b0, idx_b1)
    arows = (arows0, arows1)
    brows = (brows0, brows1)
    out_v = (out_v0, out_v1)
    sem_i = (sem_i0, sem_i1)
    sem_g = (sem_g0, sem_g1)
    sem_h = (sem_h0, sem_h1)
    sem_o = (sem_o0, sem_o1)

    lane = lax.iota(jnp.int32, _L)
    sstart = ((lane >= 1).astype(jnp.int32) + 3 * (lane >= 4).astype(jnp.int32)
              + 5 * (lane >= 9).astype(jnp.int32))
    i1 = jnp.maximum(lane - 1, 0)
    i2 = jnp.maximum(lane - 2, 0)
    i4 = jnp.maximum(lane - 4, 0)
    m1 = (lane - 1) >= sstart
    m2 = (lane - 2) >= sstart
    m4 = (lane - 4) >= sstart
    q = lane & 3
    fin = q * q + 2 * q
    grp = lane >> 2
    fzero = jnp.float32(0.0)

    def start_idx(k, b):
        e0 = base_e + k * _W
        pltpu.async_copy(row_hbm.at[pl.ds(e0, _W)], idx_a[b], sem_i[b])
        pltpu.async_copy(col_hbm.at[pl.ds(e0, _W)], idx_b[b], sem_i[b])

    def drain_idx(b):
        pltpu.make_async_copy(row_hbm.at[pl.ds(0, _W)], idx_a[b],
                              sem_i[b]).wait()
        pltpu.make_async_copy(col_hbm.at[pl.ds(0, _W)], idx_b[b],
                              sem_i[b]).wait()

    def start_gathers(b):
        for j in range(_GPC):
            pltpu.async_copy(
                ntab.at[idx_a[b].at[pl.ds(j * _G, _G)]],
                arows[b].at[pl.ds(j * _G, _G)], sem_g[b])
            pltpu.async_copy(
                node_hbm.at[idx_b[b].at[pl.ds(j * _G, _G)]],
                brows[b].at[pl.ds(j * _G, _G)], sem_h[b])

    def drain_gathers(b):
        pltpu.make_async_copy(ntab.at[pl.ds(0, _W)], arows[b],
                              sem_g[b]).wait()
        pltpu.make_async_copy(node_hbm.at[pl.ds(0, _W)], brows[b],
                              sem_h[b]).wait()

    def start_out(k, b):
        e0 = base_e + k * _W
        pltpu.async_copy(out_v[b], out_hbm.at[pl.ds(e0 * 4, _W * 4)],
                         sem_o[b])

    def drain_out(b):
        pltpu.make_async_copy(out_v[b], out_hbm.at[pl.ds(0, _W * 4)],
                              sem_o[b]).wait()

    def seg_sums(t):
        x = t + jnp.where(m1, _take16(t, i1), fzero)
        x = x + jnp.where(m2, _take16(x, i2), fzero)
        x = x + jnp.where(m4, _take16(x, i4), fzero)
        return _take16(x, fin)

    def compute(b):
        @plsc.parallel_loop(0, _W, step=4, unroll=2)
        def _edge(e):
            g = [seg_sums(arows[b][e + u, :] * brows[b][e + u, :])
                 for u in range(4)]
            acc = jnp.where(grp == 1, g[1], g[0])
            acc = jnp.where(grp == 2, g[2], acc)
            acc = jnp.where(grp == 3, g[3], acc)
            out_v[b][pl.ds(e * 4, _L)] = acc

    start_idx(0, 0)
    start_idx(1, 1)
    drain_idx(0)
    start_gathers(0)

    @pl.loop(0, n_chunks, step=2)
    def _pair(kk):
        for b in (0, 1):
            k = kk + b
            ob = 1 - b
            drain_gathers(b)

            @pl.when(k + 2 < n_chunks)
            def _():
                start_idx(k + 2, b)

            @pl.when(k + 1 < n_chunks)
            def _():
                drain_idx(ob)
                start_gathers(ob)

            @pl.when(k >= 2)
            def _():
                drain_out(b)

            compute(b)
            start_out(k, b)

    drain_out(0)
    drain_out(1)


@functools.partial(jax.jit, static_argnames=("e_pad",))
def _sc_call(node_sh, row2, col2, e_pad):
    mesh = plsc.VectorSubcoreMesh(core_axis_name="c", subcore_axis_name="s")
    cp = pltpu.CompilerParams()
    if "needs_layout_passes" in pltpu.CompilerParams.__dataclass_fields__:
        cp = dataclasses.replace(cp, needs_layout_passes=False)
    if "use_tc_tiling_on_sc" in pltpu.CompilerParams.__dataclass_fields__:
        cp = dataclasses.replace(cp, use_tc_tiling_on_sc=False)
    kfn = pl.kernel(
        _sc_kernel_body,
        out_type=jax.ShapeDtypeStruct((e_pad * 4,), jnp.float32),
        mesh=mesh,
        scratch_types=(
            [pltpu.VMEM_SHARED(node_sh.shape, jnp.float32)]
            + [pltpu.VMEM((_W,), jnp.int32)] * 4
            + [pltpu.VMEM((_W, _L), jnp.float32)] * 4
            + [pltpu.VMEM((_W * 4,), jnp.float32)] * 2
            + [pltpu.SemaphoreType.DMA] * 8
        ),
        compiler_params=cp,
    )
    return kfn(node_sh, row2, col2)


def kernel(edge_index, node_sh):
    e = edge_index.shape[1]
    per_tile = -(-e // (_NW * _W * 2)) * _W * 2
    e_pad = per_tile * _NW
    ei = edge_index.astype(jnp.int32)
    row = jnp.pad(ei[0], (0, e_pad - e))
    col = jnp.pad(ei[1], (0, e_pad - e))
    out = _sc_call(node_sh, row, col, e_pad)
    return out[:e * 4].reshape(e, 4)
